# baseline (speedup 1.0000x reference)
import jax
import jax.numpy as jnp
from jax import lax
from jax.experimental import pallas as pl
from jax.experimental.pallas import tpu as pltpu
from jax.experimental.pallas import tpu_sc as plsc

_AH = 7
_AW = 7
_NPP = _AH * _AW
_SCALE = 0.125

_B, _C, _H, _W = 4, 256, 64, 64
_N = 1000
_NC, _NS, _L = 2, 16, 16
_NWORK = _NC * _NS
_RPW = 32
_EXTRA = _N - (_RPW - 1) * _NWORK
_SLOTS = 56
_TSLOTS = _RPW * _SLOTS
_GROUPS = _C // _L
_CLEN = (16, 16, 16, 8)
_CPTS = (16, 16, 16, 1)
_OUT_W = _C * _NPP


def _sc_body(table, rois, out, rois_v,
             idx0, idx1, idx2, idx3, w0, w1, w2, w3,
             ulA, urA, dlA, drA, ulB, urB, dlB, drB,
             ovA, ovB, sem_g, sem_o):
    wid = lax.axis_index("s") * _NC + lax.axis_index("c")
    base_roi = wid * (_RPW - 1) + jnp.minimum(wid, _EXTRA)
    last_rl = _RPW - 1 - jnp.where(wid < _EXTRA, 0, 1)

    pltpu.sync_copy(rois, rois_v)

    lanes = lax.iota(jnp.int32, _L)
    lanes49 = lanes * _NPP
    idx_refs = (idx0, idx1, idx2, idx3)
    w_refs = (w0, w1, w2, w3)
    bufs = ((ulA, urA, dlA, drA), (ulB, urB, dlB, drB))
    outs = (ovA, ovB)

    def compute_meta(i, carry):
        slot = i * _L
        sv = jnp.full((_L,), slot, jnp.int32) + lanes
        rl = lax.div(sv, _SLOTS)
        within = sv - rl * _SLOTS
        pad_ok = within < _NPP
        ph = lax.div(within, _AW)
        pw = within - ph * _AW
        n5 = (jnp.minimum(rl, last_rl) + base_roi) * 5
        bf = plsc.load_gather(rois_v, [n5])
        x1 = plsc.load_gather(rois_v, [n5 + 1])
        y1 = plsc.load_gather(rois_v, [n5 + 2])
        x2 = plsc.load_gather(rois_v, [n5 + 3])
        y2 = plsc.load_gather(rois_v, [n5 + 4])
        sw = x1 * _SCALE
        sh = y1 * _SCALE
        roi_w = jnp.maximum(x2 * _SCALE - sw, 0.0)
        roi_h = jnp.maximum(y2 * _SCALE - sh, 0.0)
        bin_w = roi_w / (_AW - 1.0)
        bin_h = roi_h / (_AH - 1.0)
        hh = sh + ph.astype(jnp.float32) * bin_h
        ww = sw + pw.astype(jnp.float32) * bin_w
        valid = (hh >= 0.0) & (hh < _H) & (ww >= 0.0) & (ww < _W) & pad_ok
        hi = jnp.clip(hh.astype(jnp.int32), 0, _H - 2)
        wi = jnp.clip(ww.astype(jnp.int32), 0, _W - 2)
        hr = hh - hi.astype(jnp.float32)
        wr = ww - wi.astype(jnp.float32)
        vf = jnp.where(valid, 1.0, 0.0)
        bi = bf.astype(jnp.int32)
        base_idx = bi * (_H * _W) + hi * _W + wi
        sl = pl.ds(slot, _L)
        idx0[sl] = base_idx
        idx1[sl] = base_idx + 1
        idx2[sl] = base_idx + _W
        idx3[sl] = base_idx + _W + 1
        w0[sl] = (1.0 - hr) * (1.0 - wr) * vf
        w1[sl] = (1.0 - hr) * wr * vf
        w2[sl] = hr * (1.0 - wr) * vf
        w3[sl] = hr * wr * vf
        return carry

    lax.fori_loop(0, _TSLOTS // _L, compute_meta, 0)

    def issue_gathers(off, glen, bset):
        off = jnp.minimum(off, _TSLOTS - glen)
        for cref, dst in zip(idx_refs, bset):
            pltpu.async_copy(table.at[cref.at[pl.ds(off, glen)]],
                             dst.at[pl.ds(0, glen)], sem_g)

    def drain_gathers(glen, bset):
        for dst in bset:
            pltpu.make_async_copy(table.at[pl.ds(0, glen)],
                                  dst.at[pl.ds(0, glen)], sem_g).wait()

    def chunk_body(r, g, ov):
        bset = bufs[g & 1]
        drain_gathers(_CLEN[g], bset)

        def do_point(p, cc):
            pv = jnp.full((_L,), r * _SLOTS + g * 16 + p, jnp.int32)
            a0 = plsc.load_gather(w0, [pv])
            a1 = plsc.load_gather(w1, [pv])
            a2 = plsc.load_gather(w2, [pv])
            a3 = plsc.load_gather(w3, [pv])
            pidx = lanes49 + (g * 16 + p)
            for gg in range(_GROUPS):
                sl = pl.ds(gg * _L, _L)
                acc = (bset[0][p, sl] * a0 + bset[1][p, sl] * a1
                       + bset[2][p, sl] * a2 + bset[3][p, sl] * a3)
                plsc.store_scatter(ov, [pidx + gg * (_L * _NPP)], acc)
            return cc

        lax.fori_loop(0, _CPTS[g], do_point, 0)
        nxt = (g + 2) % 4
        nxt_r = r + (g + 2) // 4
        issue_gathers(nxt_r * _SLOTS + nxt * 16, _CLEN[nxt], bset)

    def roi_body(r, par):
        ov = outs[par]
        pltpu.make_async_copy(ov, out.at[base_roi], sem_o).wait()
        for g in range(4):
            chunk_body(r, g, ov)
        dst_row = base_roi + jnp.minimum(r, last_rl)
        pltpu.async_copy(ov, out.at[dst_row], sem_o)

    def do_pair(k, carry):
        roi_body(k * 2, 0)
        roi_body(k * 2 + 1, 1)
        return carry

    issue_gathers(0, _CLEN[0], bufs[0])
    issue_gathers(16, _CLEN[1], bufs[1])
    pltpu.async_copy(ovA, out.at[base_roi], sem_o)
    pltpu.async_copy(ovB, out.at[base_roi + 1], sem_o)

    lax.fori_loop(0, _RPW // 2, do_pair, 0)

    for bset in bufs:
        drain_gathers(16, bset)
    for ov in outs:
        pltpu.make_async_copy(ov, out.at[base_roi], sem_o).wait()


def _build_sc_call():
    cbuf = pltpu.VMEM((16, _C), jnp.float32)
    return pl.kernel(
        _sc_body,
        out_type=jax.ShapeDtypeStruct((_N, _OUT_W), jnp.float32),
        mesh=plsc.VectorSubcoreMesh(core_axis_name="c", subcore_axis_name="s"),
        compiler_params=pltpu.CompilerParams(needs_layout_passes=False),
        scratch_types=[
            pltpu.VMEM((_N * 5,), jnp.float32),
            pltpu.VMEM((_TSLOTS,), jnp.int32),
            pltpu.VMEM((_TSLOTS,), jnp.int32),
            pltpu.VMEM((_TSLOTS,), jnp.int32),
            pltpu.VMEM((_TSLOTS,), jnp.int32),
            pltpu.VMEM((_TSLOTS,), jnp.float32),
            pltpu.VMEM((_TSLOTS,), jnp.float32),
            pltpu.VMEM((_TSLOTS,), jnp.float32),
            pltpu.VMEM((_TSLOTS,), jnp.float32),
            cbuf, cbuf, cbuf, cbuf, cbuf, cbuf, cbuf, cbuf,
            pltpu.VMEM((_OUT_W,), jnp.float32),
            pltpu.VMEM((_OUT_W,), jnp.float32),
            pltpu.SemaphoreType.DMA,
            pltpu.SemaphoreType.DMA,
        ],
    )


def kernel(features, rois):
    table = jnp.transpose(features, (0, 2, 3, 1)).reshape(_B * _H * _W, _C)
    flat = _build_sc_call()(table, rois.reshape(_N * 5))
    return flat.reshape(_N, _C, _AH, _AW)


# speedup vs baseline: 1.0034x; 1.0034x over previous
import jax
import jax.numpy as jnp
from jax import lax
from jax.experimental import pallas as pl
from jax.experimental.pallas import tpu as pltpu
from jax.experimental.pallas import tpu_sc as plsc

_AH = 7
_AW = 7
_NPP = _AH * _AW
_SCALE = 0.125

_B, _C, _H, _W = 4, 256, 64, 64
_N = 1000
_NC, _NS, _L = 2, 16, 16
_NWORK = _NC * _NS
_RPW = 32
_EXTRA = _N - (_RPW - 1) * _NWORK
_SLOTS = 56
_TSLOTS = _RPW * _SLOTS
_GROUPS = _C // _L
_CLEN = (16, 16, 16, 8)
_CPTS = (16, 16, 16, 1)
_OUT_W = _C * _NPP


def _sc_body(table, rois, out, rois_v,
             idx0, idx1, idx2, idx3, w0, w1, w2, w3,
             ulA, urA, dlA, drA, ulB, urB, dlB, drB,
             ovA, ovB, sem_g, sem_o):
    wid = lax.axis_index("s") * _NC + lax.axis_index("c")
    base_roi = wid * (_RPW - 1) + jnp.minimum(wid, _EXTRA)
    last_rl = _RPW - 1 - jnp.where(wid < _EXTRA, 0, 1)

    pltpu.sync_copy(rois, rois_v)

    lanes = lax.iota(jnp.int32, _L)
    lanes49 = lanes * _NPP
    idx_refs = (idx0, idx1, idx2, idx3)
    w_refs = (w0, w1, w2, w3)
    bufs = ((ulA, urA, dlA, drA), (ulB, urB, dlB, drB))
    outs = (ovA, ovB)

    def compute_meta(i, carry):
        slot = i * _L
        sv = jnp.full((_L,), slot, jnp.int32) + lanes
        rl = lax.div(sv, _SLOTS)
        within = sv - rl * _SLOTS
        pad_ok = within < _NPP
        ph = lax.div(within, _AW)
        pw = within - ph * _AW
        n5 = (jnp.minimum(rl, last_rl) + base_roi) * 5
        bf = plsc.load_gather(rois_v, [n5])
        x1 = plsc.load_gather(rois_v, [n5 + 1])
        y1 = plsc.load_gather(rois_v, [n5 + 2])
        x2 = plsc.load_gather(rois_v, [n5 + 3])
        y2 = plsc.load_gather(rois_v, [n5 + 4])
        sw = x1 * _SCALE
        sh = y1 * _SCALE
        roi_w = jnp.maximum(x2 * _SCALE - sw, 0.0)
        roi_h = jnp.maximum(y2 * _SCALE - sh, 0.0)
        bin_w = roi_w / (_AW - 1.0)
        bin_h = roi_h / (_AH - 1.0)
        hh = sh + ph.astype(jnp.float32) * bin_h
        ww = sw + pw.astype(jnp.float32) * bin_w
        valid = (hh >= 0.0) & (hh < _H) & (ww >= 0.0) & (ww < _W) & pad_ok
        hi = jnp.clip(hh.astype(jnp.int32), 0, _H - 2)
        wi = jnp.clip(ww.astype(jnp.int32), 0, _W - 2)
        hr = hh - hi.astype(jnp.float32)
        wr = ww - wi.astype(jnp.float32)
        vf = jnp.where(valid, 1.0, 0.0)
        bi = bf.astype(jnp.int32)
        base_idx = bi * (_H * _W) + hi * _W + wi
        sl = pl.ds(slot, _L)
        idx0[sl] = base_idx
        idx1[sl] = base_idx + 1
        idx2[sl] = base_idx + _W
        idx3[sl] = base_idx + _W + 1
        w0[sl] = (1.0 - hr) * (1.0 - wr) * vf
        w1[sl] = (1.0 - hr) * wr * vf
        w2[sl] = hr * (1.0 - wr) * vf
        w3[sl] = hr * wr * vf
        return carry

    lax.fori_loop(0, _TSLOTS // _L, compute_meta, 0)

    def issue_gathers(off, glen, bset):
        off = jnp.minimum(off, _TSLOTS - glen)
        for cref, dst in zip(idx_refs, bset):
            pltpu.async_copy(table.at[cref.at[pl.ds(off, glen)]],
                             dst.at[pl.ds(0, glen)], sem_g)

    def drain_gathers(glen, bset):
        for dst in bset:
            pltpu.make_async_copy(table.at[pl.ds(0, glen)],
                                  dst.at[pl.ds(0, glen)], sem_g).wait()

    def chunk_body(r, g, ov):
        bset = bufs[g & 1]
        drain_gathers(_CLEN[g], bset)
        wbase = pl.ds(r * _SLOTS + g * 16, _L)
        wv0 = w0[wbase]
        wv1 = w1[wbase]
        wv2 = w2[wbase]
        wv3 = w3[wbase]

        def do_point(p, cc):
            pv = jnp.full((_L,), p, jnp.int32)
            a0 = wv0.at[pv].get(mode="promise_in_bounds")
            a1 = wv1.at[pv].get(mode="promise_in_bounds")
            a2 = wv2.at[pv].get(mode="promise_in_bounds")
            a3 = wv3.at[pv].get(mode="promise_in_bounds")
            pidx = lanes49 + (g * 16 + p)
            for gg in range(_GROUPS):
                sl = pl.ds(gg * _L, _L)
                acc = (bset[0][p, sl] * a0 + bset[1][p, sl] * a1
                       + bset[2][p, sl] * a2 + bset[3][p, sl] * a3)
                plsc.store_scatter(ov, [pidx + gg * (_L * _NPP)], acc)
            return cc

        lax.fori_loop(0, _CPTS[g], do_point, 0)
        nxt = (g + 2) % 4
        nxt_r = r + (g + 2) // 4
        issue_gathers(nxt_r * _SLOTS + nxt * 16, _CLEN[nxt], bset)

    def roi_body(r, par):
        ov = outs[par]
        pltpu.make_async_copy(ov, out.at[base_roi], sem_o).wait()
        for g in range(4):
            chunk_body(r, g, ov)
        dst_row = base_roi + jnp.minimum(r, last_rl)
        pltpu.async_copy(ov, out.at[dst_row], sem_o)

    def do_pair(k, carry):
        roi_body(k * 2, 0)
        roi_body(k * 2 + 1, 1)
        return carry

    issue_gathers(0, _CLEN[0], bufs[0])
    issue_gathers(16, _CLEN[1], bufs[1])
    pltpu.async_copy(ovA, out.at[base_roi], sem_o)
    pltpu.async_copy(ovB, out.at[base_roi + 1], sem_o)

    lax.fori_loop(0, _RPW // 2, do_pair, 0)

    for bset in bufs:
        drain_gathers(16, bset)
    for ov in outs:
        pltpu.make_async_copy(ov, out.at[base_roi], sem_o).wait()


def _build_sc_call():
    cbuf = pltpu.VMEM((16, _C), jnp.float32)
    return pl.kernel(
        _sc_body,
        out_type=jax.ShapeDtypeStruct((_N, _OUT_W), jnp.float32),
        mesh=plsc.VectorSubcoreMesh(core_axis_name="c", subcore_axis_name="s"),
        compiler_params=pltpu.CompilerParams(needs_layout_passes=False),
        scratch_types=[
            pltpu.VMEM((_N * 5,), jnp.float32),
            pltpu.VMEM((_TSLOTS,), jnp.int32),
            pltpu.VMEM((_TSLOTS,), jnp.int32),
            pltpu.VMEM((_TSLOTS,), jnp.int32),
            pltpu.VMEM((_TSLOTS,), jnp.int32),
            pltpu.VMEM((_TSLOTS + 8,), jnp.float32),
            pltpu.VMEM((_TSLOTS + 8,), jnp.float32),
            pltpu.VMEM((_TSLOTS + 8,), jnp.float32),
            pltpu.VMEM((_TSLOTS + 8,), jnp.float32),
            cbuf, cbuf, cbuf, cbuf, cbuf, cbuf, cbuf, cbuf,
            pltpu.VMEM((_OUT_W,), jnp.float32),
            pltpu.VMEM((_OUT_W,), jnp.float32),
            pltpu.SemaphoreType.DMA,
            pltpu.SemaphoreType.DMA,
        ],
    )


def kernel(features, rois):
    table = jnp.transpose(features, (0, 2, 3, 1)).reshape(_B * _H * _W, _C)
    flat = _build_sc_call()(table, rois.reshape(_N * 5))
    return flat.reshape(_N, _C, _AH, _AW)


# speedup vs baseline: 1.0874x; 1.0837x over previous
import jax
import jax.numpy as jnp
from jax import lax
from jax.experimental import pallas as pl
from jax.experimental.pallas import tpu as pltpu
from jax.experimental.pallas import tpu_sc as plsc

_AH = 7
_AW = 7
_NPP = _AH * _AW
_SCALE = 0.125

_B, _C, _H, _W = 4, 256, 64, 64
_N = 1000
_NC, _NS, _L = 2, 16, 16
_NWORK = _NC * _NS
_RPW = 32
_EXTRA = _N - (_RPW - 1) * _NWORK
_SLOTS = 56
_TSLOTS = _RPW * _SLOTS
_GROUPS = _C // _L
_CLEN = (16, 16, 16, 8)
_CPTS = (16, 16, 16, 1)
_OUT_W = _C * _NPP


def _sc_body(table, rois, out, rois_v,
             idx0, idx1, idx2, idx3, w0, w1, w2, w3,
             ulA, urA, dlA, drA, ulB, urB, dlB, drB,
             ovA, ovB, sem_g, sem_o):
    wid = lax.axis_index("s") * _NC + lax.axis_index("c")
    base_roi = wid * (_RPW - 1) + jnp.minimum(wid, _EXTRA)
    last_rl = _RPW - 1 - jnp.where(wid < _EXTRA, 0, 1)

    pltpu.sync_copy(rois, rois_v)

    lanes = lax.iota(jnp.int32, _L)
    lanes49 = lanes * _NPP
    idx_refs = (idx0, idx1, idx2, idx3)
    w_refs = (w0, w1, w2, w3)
    bufs = ((ulA, urA, dlA, drA), (ulB, urB, dlB, drB))
    outs = (ovA, ovB)

    def compute_meta(i, carry):
        slot = i * _L
        sv = jnp.full((_L,), slot, jnp.int32) + lanes
        rl = lax.div(sv, _SLOTS)
        within = sv - rl * _SLOTS
        pad_ok = within < _NPP
        ph = lax.div(within, _AW)
        pw = within - ph * _AW
        n5 = (jnp.minimum(rl, last_rl) + base_roi) * 5
        bf = plsc.load_gather(rois_v, [n5])
        x1 = plsc.load_gather(rois_v, [n5 + 1])
        y1 = plsc.load_gather(rois_v, [n5 + 2])
        x2 = plsc.load_gather(rois_v, [n5 + 3])
        y2 = plsc.load_gather(rois_v, [n5 + 4])
        sw = x1 * _SCALE
        sh = y1 * _SCALE
        roi_w = jnp.maximum(x2 * _SCALE - sw, 0.0)
        roi_h = jnp.maximum(y2 * _SCALE - sh, 0.0)
        bin_w = roi_w / (_AW - 1.0)
        bin_h = roi_h / (_AH - 1.0)
        hh = sh + ph.astype(jnp.float32) * bin_h
        ww = sw + pw.astype(jnp.float32) * bin_w
        valid = (hh >= 0.0) & (hh < _H) & (ww >= 0.0) & (ww < _W) & pad_ok
        hi = jnp.clip(hh.astype(jnp.int32), 0, _H - 2)
        wi = jnp.clip(ww.astype(jnp.int32), 0, _W - 2)
        hr = hh - hi.astype(jnp.float32)
        wr = ww - wi.astype(jnp.float32)
        vf = jnp.where(valid, 1.0, 0.0)
        bi = bf.astype(jnp.int32)
        base_idx = bi * (_H * _W) + hi * _W + wi
        sl = pl.ds(slot, _L)
        idx0[sl] = base_idx
        idx1[sl] = base_idx + 1
        idx2[sl] = base_idx + _W
        idx3[sl] = base_idx + _W + 1
        w0[sl] = (1.0 - hr) * (1.0 - wr) * vf
        w1[sl] = (1.0 - hr) * wr * vf
        w2[sl] = hr * (1.0 - wr) * vf
        w3[sl] = hr * wr * vf
        return carry

    lax.fori_loop(0, _TSLOTS // _L, compute_meta, 0)

    def issue_gathers(off, glen, bset):
        off = jnp.minimum(off, _TSLOTS - glen)
        for cref, dst in zip(idx_refs, bset):
            pltpu.async_copy(table.at[cref.at[pl.ds(off, glen)]],
                             dst.at[pl.ds(0, glen)], sem_g)

    def drain_gathers(glen, bset):
        for dst in bset:
            pltpu.make_async_copy(table.at[pl.ds(0, glen)],
                                  dst.at[pl.ds(0, glen)], sem_g).wait()

    perm_idx = [lanes ^ s for s in (1, 2, 4, 8)]
    stage_msk = [(lanes & s) == 0 for s in (1, 2, 4, 8)]

    def chunk_body(r, g, ov):
        bset = bufs[g & 1]
        drain_gathers(_CLEN[g], bset)
        wbase = pl.ds(r * _SLOTS + g * 16, _L)
        wv0 = w0[wbase]
        wv1 = w1[wbase]
        wv2 = w2[wbase]
        wv3 = w3[wbase]

        def splat(wv, p):
            return wv.at[jnp.full((_L,), p, jnp.int32)].get(
                mode="promise_in_bounds")

        if _CPTS[g] == _L:
            def do_group(gg, cc):
                cbase = gg * _L
                vs = []
                for p in range(_L):
                    a0 = splat(wv0, p)
                    a1 = splat(wv1, p)
                    a2 = splat(wv2, p)
                    a3 = splat(wv3, p)
                    sl = pl.ds(cbase, _L)
                    vs.append(bset[0][p, sl] * a0 + bset[1][p, sl] * a1
                              + bset[2][p, sl] * a2 + bset[3][p, sl] * a3)
                for s in range(4):
                    step = 1 << s
                    pi = perm_idx[s]
                    mk = stage_msk[s]
                    nvs = list(vs)
                    for i in range(_L):
                        if i & step == 0:
                            j = i | step
                            pa = vs[i].at[pi].get(mode="promise_in_bounds")
                            pb = vs[j].at[pi].get(mode="promise_in_bounds")
                            nvs[i] = jnp.where(mk, vs[i], pb)
                            nvs[j] = jnp.where(mk, pa, vs[j])
                    vs = nvs
                obase = (cbase * _NPP) + g * 16
                for c in range(_L):
                    ov[pl.ds(obase + c * _NPP, _L)] = vs[c]
                return cc

            lax.fori_loop(0, _GROUPS, do_group, 0)
        else:
            def do_point(p, cc):
                a0 = splat(wv0, p)
                a1 = splat(wv1, p)
                a2 = splat(wv2, p)
                a3 = splat(wv3, p)
                pidx = lanes49 + (g * 16 + p)
                for gg in range(_GROUPS):
                    sl = pl.ds(gg * _L, _L)
                    acc = (bset[0][p, sl] * a0 + bset[1][p, sl] * a1
                           + bset[2][p, sl] * a2 + bset[3][p, sl] * a3)
                    plsc.store_scatter(ov, [pidx + gg * (_L * _NPP)], acc)
                return cc

            lax.fori_loop(0, _CPTS[g], do_point, 0)
        nxt = (g + 2) % 4
        nxt_r = r + (g + 2) // 4
        issue_gathers(nxt_r * _SLOTS + nxt * 16, _CLEN[nxt], bset)

    def roi_body(r, par):
        ov = outs[par]
        pltpu.make_async_copy(ov, out.at[base_roi], sem_o).wait()
        for g in range(4):
            chunk_body(r, g, ov)
        dst_row = base_roi + jnp.minimum(r, last_rl)
        pltpu.async_copy(ov, out.at[dst_row], sem_o)

    def do_pair(k, carry):
        roi_body(k * 2, 0)
        roi_body(k * 2 + 1, 1)
        return carry

    issue_gathers(0, _CLEN[0], bufs[0])
    issue_gathers(16, _CLEN[1], bufs[1])
    pltpu.async_copy(ovA, out.at[base_roi], sem_o)
    pltpu.async_copy(ovB, out.at[base_roi + 1], sem_o)

    lax.fori_loop(0, _RPW // 2, do_pair, 0)

    for bset in bufs:
        drain_gathers(16, bset)
    for ov in outs:
        pltpu.make_async_copy(ov, out.at[base_roi], sem_o).wait()


def _build_sc_call():
    cbuf = pltpu.VMEM((16, _C), jnp.float32)
    return pl.kernel(
        _sc_body,
        out_type=jax.ShapeDtypeStruct((_N, _OUT_W), jnp.float32),
        mesh=plsc.VectorSubcoreMesh(core_axis_name="c", subcore_axis_name="s"),
        compiler_params=pltpu.CompilerParams(needs_layout_passes=False),
        scratch_types=[
            pltpu.VMEM((_N * 5,), jnp.float32),
            pltpu.VMEM((_TSLOTS,), jnp.int32),
            pltpu.VMEM((_TSLOTS,), jnp.int32),
            pltpu.VMEM((_TSLOTS,), jnp.int32),
            pltpu.VMEM((_TSLOTS,), jnp.int32),
            pltpu.VMEM((_TSLOTS + 8,), jnp.float32),
            pltpu.VMEM((_TSLOTS + 8,), jnp.float32),
            pltpu.VMEM((_TSLOTS + 8,), jnp.float32),
            pltpu.VMEM((_TSLOTS + 8,), jnp.float32),
            cbuf, cbuf, cbuf, cbuf, cbuf, cbuf, cbuf, cbuf,
            pltpu.VMEM((_OUT_W,), jnp.float32),
            pltpu.VMEM((_OUT_W,), jnp.float32),
            pltpu.SemaphoreType.DMA,
            pltpu.SemaphoreType.DMA,
        ],
    )


def kernel(features, rois):
    table = jnp.transpose(features, (0, 2, 3, 1)).reshape(_B * _H * _W, _C)
    flat = _build_sc_call()(table, rois.reshape(_N * 5))
    return flat.reshape(_N, _C, _AH, _AW)


# speedup vs baseline: 1.1684x; 1.0745x over previous
import jax
import jax.numpy as jnp
from jax import lax
from jax.experimental import pallas as pl
from jax.experimental.pallas import tpu as pltpu
from jax.experimental.pallas import tpu_sc as plsc

_AH = 7
_AW = 7
_NPP = _AH * _AW
_SCALE = 0.125

_B, _C, _H, _W = 4, 256, 64, 64
_N = 1000
_NC, _NS, _L = 2, 16, 16
_NWORK = _NC * _NS
_RPW = 32
_EXTRA = _N - (_RPW - 1) * _NWORK
_SLOTS = 56
_TSLOTS = _RPW * _SLOTS
_GROUPS = _C // _L
_CLEN = (32, 24)
_NBLK = (2, 1)
_OUT_W = _C * _NPP


def _sc_body(table, rois, out, rois_v,
             idx0, idx1, idx2, idx3, w0, w1, w2, w3,
             ulA, urA, dlA, drA, ulB, urB, dlB, drB,
             ovA, ovB, sem_g, sem_o):
    wid = lax.axis_index("s") * _NC + lax.axis_index("c")
    base_roi = wid * (_RPW - 1) + jnp.minimum(wid, _EXTRA)
    last_rl = _RPW - 1 - jnp.where(wid < _EXTRA, 0, 1)

    pltpu.sync_copy(rois, rois_v)

    lanes = lax.iota(jnp.int32, _L)
    lanes49 = lanes * _NPP
    idx_refs = (idx0, idx1, idx2, idx3)
    w_refs = (w0, w1, w2, w3)
    bufs = ((ulA, urA, dlA, drA), (ulB, urB, dlB, drB))
    outs = (ovA, ovB)

    def compute_meta(i, carry):
        slot = i * _L
        sv = jnp.full((_L,), slot, jnp.int32) + lanes
        rl = lax.div(sv, _SLOTS)
        within = sv - rl * _SLOTS
        pad_ok = within < _NPP
        ph = lax.div(within, _AW)
        pw = within - ph * _AW
        n5 = (jnp.minimum(rl, last_rl) + base_roi) * 5
        bf = plsc.load_gather(rois_v, [n5])
        x1 = plsc.load_gather(rois_v, [n5 + 1])
        y1 = plsc.load_gather(rois_v, [n5 + 2])
        x2 = plsc.load_gather(rois_v, [n5 + 3])
        y2 = plsc.load_gather(rois_v, [n5 + 4])
        sw = x1 * _SCALE
        sh = y1 * _SCALE
        roi_w = jnp.maximum(x2 * _SCALE - sw, 0.0)
        roi_h = jnp.maximum(y2 * _SCALE - sh, 0.0)
        bin_w = roi_w / (_AW - 1.0)
        bin_h = roi_h / (_AH - 1.0)
        hh = sh + ph.astype(jnp.float32) * bin_h
        ww = sw + pw.astype(jnp.float32) * bin_w
        valid = (hh >= 0.0) & (hh < _H) & (ww >= 0.0) & (ww < _W) & pad_ok
        hi = jnp.clip(hh.astype(jnp.int32), 0, _H - 2)
        wi = jnp.clip(ww.astype(jnp.int32), 0, _W - 2)
        hr = hh - hi.astype(jnp.float32)
        wr = ww - wi.astype(jnp.float32)
        vf = jnp.where(valid, 1.0, 0.0)
        bi = bf.astype(jnp.int32)
        base_idx = bi * (_H * _W) + hi * _W + wi
        sl = pl.ds(slot, _L)
        idx0[sl] = base_idx
        idx1[sl] = base_idx + 1
        idx2[sl] = base_idx + _W
        idx3[sl] = base_idx + _W + 1
        w0[sl] = (1.0 - hr) * (1.0 - wr) * vf
        w1[sl] = (1.0 - hr) * wr * vf
        w2[sl] = hr * (1.0 - wr) * vf
        w3[sl] = hr * wr * vf
        return carry

    lax.fori_loop(0, _TSLOTS // _L, compute_meta, 0)

    def issue_gathers(off, glen, bset):
        off = jnp.minimum(off, _TSLOTS - glen)
        for cref, dst in zip(idx_refs, bset):
            pltpu.async_copy(table.at[cref.at[pl.ds(off, glen)]],
                             dst.at[pl.ds(0, glen)], sem_g)

    def drain_gathers(glen, bset):
        for dst in bset:
            pltpu.make_async_copy(table.at[pl.ds(0, glen)],
                                  dst.at[pl.ds(0, glen)], sem_g).wait()

    perm_idx = [lanes ^ s for s in (1, 2, 4, 8)]
    stage_msk = [(lanes & s) == 0 for s in (1, 2, 4, 8)]

    def splat(wv, p):
        return wv.at[jnp.full((_L,), p, jnp.int32)].get(
            mode="promise_in_bounds")

    def block_body(bset, ov, slot_base, row_base, pt_base):
        wbase = pl.ds(slot_base, _L)
        wv0 = w0[wbase]
        wv1 = w1[wbase]
        wv2 = w2[wbase]
        wv3 = w3[wbase]

        def do_group(gg, cc):
            cbase = gg * _L
            sl = pl.ds(cbase, _L)
            vs = []
            for p in range(_L):
                row = row_base + p
                vs.append(bset[0][row, sl] * splat(wv0, p)
                          + bset[1][row, sl] * splat(wv1, p)
                          + bset[2][row, sl] * splat(wv2, p)
                          + bset[3][row, sl] * splat(wv3, p))
            for s in range(4):
                step = 1 << s
                pi = perm_idx[s]
                mk = stage_msk[s]
                nvs = list(vs)
                for i in range(_L):
                    if i & step == 0:
                        j = i | step
                        pa = vs[i].at[pi].get(mode="promise_in_bounds")
                        pb = vs[j].at[pi].get(mode="promise_in_bounds")
                        nvs[i] = jnp.where(mk, vs[i], pb)
                        nvs[j] = jnp.where(mk, pa, vs[j])
                vs = nvs
            obase = (cbase * _NPP) + pt_base
            for c in range(_L):
                ov[pl.ds(obase + c * _NPP, _L)] = vs[c]
            return cc

        lax.fori_loop(0, _GROUPS, do_group, 0)

    def tail_body(bset, ov, slot_base, row, pt):
        wbase = pl.ds(slot_base, _L)
        a0 = splat(w0[wbase], 0)
        a1 = splat(w1[wbase], 0)
        a2 = splat(w2[wbase], 0)
        a3 = splat(w3[wbase], 0)

        def do_group(gg, cc):
            sl = pl.ds(gg * _L, _L)
            acc = (bset[0][row, sl] * a0 + bset[1][row, sl] * a1
                   + bset[2][row, sl] * a2 + bset[3][row, sl] * a3)
            plsc.store_scatter(ov, [lanes49 + pt + gg * (_L * _NPP)], acc)
            return cc

        lax.fori_loop(0, _GROUPS, do_group, 0)

    def chunk_body(r, g, ov):
        bset = bufs[g]
        drain_gathers(_CLEN[g], bset)
        for blk in range(_NBLK[g]):
            block_body(bset, ov, r * _SLOTS + g * 32 + blk * 16,
                       blk * 16, g * 32 + blk * 16)
        if g == 1:
            tail_body(bset, ov, r * _SLOTS + 48, 16, 48)
        issue_gathers((r + 1) * _SLOTS + g * 32, _CLEN[g], bset)

    def roi_body(r, par):
        ov = outs[par]
        pltpu.make_async_copy(ov, out.at[base_roi], sem_o).wait()
        for g in range(2):
            chunk_body(r, g, ov)
        dst_row = base_roi + jnp.minimum(r, last_rl)
        pltpu.async_copy(ov, out.at[dst_row], sem_o)

    def do_pair(k, carry):
        roi_body(k * 2, 0)
        roi_body(k * 2 + 1, 1)
        return carry

    issue_gathers(0, _CLEN[0], bufs[0])
    issue_gathers(32, _CLEN[1], bufs[1])
    pltpu.async_copy(ovA, out.at[base_roi], sem_o)
    pltpu.async_copy(ovB, out.at[base_roi + 1], sem_o)

    lax.fori_loop(0, _RPW // 2, do_pair, 0)

    for g in range(2):
        drain_gathers(_CLEN[g], bufs[g])
    for ov in outs:
        pltpu.make_async_copy(ov, out.at[base_roi], sem_o).wait()


def _build_sc_call():
    cbuf = pltpu.VMEM((32, _C), jnp.float32)
    return pl.kernel(
        _sc_body,
        out_type=jax.ShapeDtypeStruct((_N, _OUT_W), jnp.float32),
        mesh=plsc.VectorSubcoreMesh(core_axis_name="c", subcore_axis_name="s"),
        compiler_params=pltpu.CompilerParams(needs_layout_passes=False),
        scratch_types=[
            pltpu.VMEM((_N * 5,), jnp.float32),
            pltpu.VMEM((_TSLOTS,), jnp.int32),
            pltpu.VMEM((_TSLOTS,), jnp.int32),
            pltpu.VMEM((_TSLOTS,), jnp.int32),
            pltpu.VMEM((_TSLOTS,), jnp.int32),
            pltpu.VMEM((_TSLOTS + 8,), jnp.float32),
            pltpu.VMEM((_TSLOTS + 8,), jnp.float32),
            pltpu.VMEM((_TSLOTS + 8,), jnp.float32),
            pltpu.VMEM((_TSLOTS + 8,), jnp.float32),
            cbuf, cbuf, cbuf, cbuf, cbuf, cbuf, cbuf, cbuf,
            pltpu.VMEM((_OUT_W,), jnp.float32),
            pltpu.VMEM((_OUT_W,), jnp.float32),
            pltpu.SemaphoreType.DMA,
            pltpu.SemaphoreType.DMA,
        ],
    )


def kernel(features, rois):
    table = jnp.transpose(features, (0, 2, 3, 1)).reshape(_B * _H * _W, _C)
    flat = _build_sc_call()(table, rois.reshape(_N * 5))
    return flat.reshape(_N, _C, _AH, _AW)
